# baseline (device time: 36058 ns/iter reference)
import jax
import jax.numpy as jnp
from jax import lax
from jax.experimental import pallas as pl
from jax.experimental.pallas import tpu as pltpu

N_DEV = 4


def kernel(x, Wq, K_ext, V_ext, Wo):
    B, Sq, D = x.shape
    Skv = K_ext.shape[1]
    Hq = K_ext.shape[2]
    Dh = K_ext.shape[3]
    Hl = Hq // N_DEV
    assert B == 2
    assert Wq.shape == (D, Hl * Dh)
    assert Wo.shape == (Hl * Dh, D)
    H2 = Sq // 2

    my = lax.axis_index("i")
    K_loc = lax.dynamic_slice_in_dim(
        K_ext.reshape(B, Skv, Hq * Dh), my * Hl * Dh, Hl * Dh, axis=2)
    V_loc = lax.dynamic_slice_in_dim(
        V_ext.reshape(B, Skv, Hq * Dh), my * Hl * Dh, Hl * Dh, axis=2)

    def body(x_ref, wq_ref, k_ref, v_ref, wo_ref, out_ref,
             ctx_ref, bias_ref, cb_ref, r1_ref, r2_ref,
             send_sems, recv_sems):
        pos = lax.axis_index("i")
        left = lax.rem(pos + N_DEV - 1, N_DEV)
        right = lax.rem(pos + 1, N_DEV)

        barrier_sem = pltpu.get_barrier_semaphore()
        for nbr in (left, right):
            pl.semaphore_signal(
                barrier_sem, inc=1,
                device_id=(nbr,), device_id_type=pl.DeviceIdType.MESH,
            )
        ri = lax.broadcasted_iota(jnp.int32, (Sq, Skv), 0)
        ci = lax.broadcasted_iota(jnp.int32, (Sq, Skv), 1)
        bias_ref[...] = jnp.where(
            ((ri // 64) % 4) == ((ci // 64) % 4), 0.0, -1e9
        ).astype(jnp.float32)
        pl.semaphore_wait(barrier_sem, 2)

        hi = pos // 2
        gray = (pos + hi) % 2
        x1 = jnp.bitwise_xor(pos, 1)
        x3 = jnp.bitwise_xor(pos, 3)
        roles = {0: (gray, x1, x3), 1: (gray, x1, x3)}

        def compute_half(b, roff, write_cb):
            qh = jnp.dot(x_ref[b, pl.ds(roff, H2), :], wq_ref[...],
                         preferred_element_type=jnp.float32)
            for h in range(Hl):
                q = qh[:, h * Dh:(h + 1) * Dh]
                k = k_ref[b][:, h * Dh:(h + 1) * Dh]
                s = lax.dot_general(
                    q, k, (((1,), (1,)), ((), ())),
                    preferred_element_type=jnp.float32)
                w = jnp.exp(s * 0.125 + bias_ref[pl.ds(roff, H2), :])
                rs = 1.0 / jnp.sum(w, axis=1, keepdims=True)
                ctx_ref[:, h * Dh:(h + 1) * Dh] = rs * jnp.dot(
                    w, v_ref[b][:, h * Dh:(h + 1) * Dh],
                    preferred_element_type=jnp.float32)
            res = jnp.dot(ctx_ref[...], wo_ref[...],
                          preferred_element_type=jnp.float32)
            out_ref[b, pl.ds(roff, H2), :] = res
            if write_cb:
                cb_ref[b, pl.ds(roff, H2), :] = res.astype(jnp.bfloat16)

        def start(bf, st, partner, off, dst):
            rdma = pltpu.make_async_remote_copy(
                src_ref=cb_ref.at[bf, pl.ds(off, H2)],
                dst_ref=dst,
                send_sem=send_sems.at[bf, st],
                recv_sem=recv_sems.at[bf, st],
                device_id=(partner,),
                device_id_type=pl.DeviceIdType.MESH,
            )
            rdma.start()
            return rdma

        def s1_start(bf):
            ka, p1, p2 = roles[bf]
            return start(bf, 0, p1, (1 - ka) * H2, r1_ref.at[bf])

        def s1_fin(bf, rdma):
            ka, _, _ = roles[bf]
            rdma.wait()
            koff = ka * H2
            acc = out_ref[bf, pl.ds(koff, H2), :] + r1_ref[bf].astype(
                jnp.float32)
            out_ref[bf, pl.ds(koff, H2), :] = acc
            cb_ref[bf, pl.ds(koff, H2), :] = acc.astype(jnp.bfloat16)

        def s2_start(bf):
            ka, p1, p2 = roles[bf]
            return start(bf, 1, p2, ka * H2, r2_ref.at[bf])

        def s2_fin(bf, rdma):
            ka, _, _ = roles[bf]
            rdma.wait()
            koff = ka * H2
            acc = out_ref[bf, pl.ds(koff, H2), :] + r2_ref[bf].astype(
                jnp.float32)
            out_ref[bf, pl.ds(koff, H2), :] = acc
            cb_ref[bf, pl.ds(koff, H2), :] = acc.astype(jnp.bfloat16)

        def s3_start(bf):
            ka, p1, p2 = roles[bf]
            off = ka * H2
            return start(bf, 2, p1, off, cb_ref.at[bf, pl.ds(off, H2)])

        def s3_fin(bf, rdma):
            ka, _, _ = roles[bf]
            rdma.wait()
            off = (1 - ka) * H2
            out_ref[bf, pl.ds(off, H2), :] = cb_ref[
                bf, pl.ds(off, H2), :].astype(jnp.float32)

        ka_a, _, _ = roles[0]
        ka_b, _, _ = roles[1]
        compute_half(0, (1 - ka_a) * H2, True)
        a1 = s1_start(0)
        compute_half(1, (1 - ka_b) * H2, True)
        b1 = s1_start(1)
        compute_half(0, ka_a * H2, False)
        s1_fin(0, a1)
        a2 = s2_start(0)
        compute_half(1, ka_b * H2, False)
        s1_fin(1, b1)
        b2 = s2_start(1)
        s2_fin(0, a2)
        a3 = s3_start(0)
        s2_fin(1, b2)
        b3 = s3_start(1)
        s3_fin(0, a3)
        s3_fin(1, b3)

    return pl.pallas_call(
        body,
        out_shape=jax.ShapeDtypeStruct((B, Sq, D), jnp.float32),
        in_specs=[pl.BlockSpec(memory_space=pltpu.VMEM)] * 5,
        out_specs=pl.BlockSpec(memory_space=pltpu.VMEM),
        scratch_shapes=[
            pltpu.VMEM((H2, Hl * Dh), jnp.float32),
            pltpu.VMEM((Sq, Skv), jnp.float32),
            pltpu.VMEM((B, Sq, D), jnp.bfloat16),
            pltpu.VMEM((B, H2, D), jnp.bfloat16),
            pltpu.VMEM((B, H2, D), jnp.bfloat16),
            pltpu.SemaphoreType.DMA((B, 3)),
            pltpu.SemaphoreType.DMA((B, 3)),
        ],
        compiler_params=pltpu.CompilerParams(collective_id=0),
    )(x, Wq, K_loc, V_loc, Wo)


# device time: 33502 ns/iter; 1.0763x vs baseline; 1.0763x over previous
import jax
import jax.numpy as jnp
from jax import lax
from jax.experimental import pallas as pl
from jax.experimental.pallas import tpu as pltpu

N_DEV = 4


def kernel(x, Wq, K_ext, V_ext, Wo):
    B, Sq, D = x.shape
    Skv = K_ext.shape[1]
    Hq = K_ext.shape[2]
    Dh = K_ext.shape[3]
    Hl = Hq // N_DEV
    assert B == 2
    assert Wq.shape == (D, Hl * Dh)
    assert Wo.shape == (Hl * Dh, D)
    H2 = Sq // 2

    my = lax.axis_index("i")
    K_loc = lax.dynamic_slice_in_dim(
        K_ext.reshape(B, Skv, Hq * Dh), my * Hl * Dh, Hl * Dh, axis=2)
    V_loc = lax.dynamic_slice_in_dim(
        V_ext.reshape(B, Skv, Hq * Dh), my * Hl * Dh, Hl * Dh, axis=2)

    def body(x_ref, wq_ref, k_ref, v_ref, wo_ref, out_ref,
             ctx_ref, bias_ref, cb_ref, r1_ref, r2_ref,
             send_sems, recv_sems):
        pos = lax.axis_index("i")
        left = lax.rem(pos + N_DEV - 1, N_DEV)
        right = lax.rem(pos + 1, N_DEV)

        barrier_sem = pltpu.get_barrier_semaphore()
        for nbr in (left, right):
            pl.semaphore_signal(
                barrier_sem, inc=1,
                device_id=(nbr,), device_id_type=pl.DeviceIdType.MESH,
            )
        ri = lax.broadcasted_iota(jnp.int32, (Sq, Skv), 0)
        ci = lax.broadcasted_iota(jnp.int32, (Sq, Skv), 1)
        bias_ref[...] = jnp.where(
            ((ri // 64) % 4) == ((ci // 64) % 4), 0.0, -1e9
        ).astype(jnp.float32)
        pl.semaphore_wait(barrier_sem, 2)

        hi = pos // 2
        gray = (pos + hi) % 2
        x1 = jnp.bitwise_xor(pos, 1)
        x3 = jnp.bitwise_xor(pos, 3)
        roles = {0: (gray, x1, x3), 1: (hi, x3, x1)}

        def compute_half(b, roff, write_cb):
            qh = jnp.dot(x_ref[b, pl.ds(roff, H2), :], wq_ref[...],
                         preferred_element_type=jnp.float32)
            for h in range(Hl):
                q = qh[:, h * Dh:(h + 1) * Dh]
                k = k_ref[b][:, h * Dh:(h + 1) * Dh]
                s = lax.dot_general(
                    q, k, (((1,), (1,)), ((), ())),
                    preferred_element_type=jnp.float32)
                w = jnp.exp(s * 0.125 + bias_ref[pl.ds(roff, H2), :])
                rs = 1.0 / jnp.sum(w, axis=1, keepdims=True)
                ctx_ref[:, h * Dh:(h + 1) * Dh] = rs * jnp.dot(
                    w, v_ref[b][:, h * Dh:(h + 1) * Dh],
                    preferred_element_type=jnp.float32)
            res = jnp.dot(ctx_ref[...], wo_ref[...],
                          preferred_element_type=jnp.float32)
            out_ref[b, pl.ds(roff, H2), :] = res
            if write_cb:
                cb_ref[b, pl.ds(roff, H2), :] = res.astype(jnp.bfloat16)

        H4 = H2 // 2

        def start(bf, st, partner, off, size, dst):
            rdma = pltpu.make_async_remote_copy(
                src_ref=cb_ref.at[bf, pl.ds(off, size)],
                dst_ref=dst,
                send_sem=send_sems.at[bf, st],
                recv_sem=recv_sems.at[bf, st],
                device_id=(partner,),
                device_id_type=pl.DeviceIdType.MESH,
            )
            rdma.start()
            return rdma

        def s1_start(bf):
            ka, p1, p2 = roles[bf]
            return start(bf, 0, p1, (1 - ka) * H2, H2, r1_ref.at[bf])

        def s1_fin(bf, rdma):
            ka, _, _ = roles[bf]
            rdma.wait()
            koff = ka * H2
            acc = out_ref[bf, pl.ds(koff, H2), :] + r1_ref[bf].astype(
                jnp.float32)
            out_ref[bf, pl.ds(koff, H2), :] = acc
            cb_ref[bf, pl.ds(koff, H2), :] = acc.astype(jnp.bfloat16)

        def s2_start(bf, j):
            ka, p1, p2 = roles[bf]
            return start(bf, 1 + j, p2, ka * H2 + j * H4, H4,
                         r2_ref.at[bf, pl.ds(j * H4, H4)])

        def s2_fin(bf, j, rdma):
            ka, _, _ = roles[bf]
            rdma.wait()
            koff = ka * H2 + j * H4
            acc = out_ref[bf, pl.ds(koff, H4), :] + r2_ref[
                bf, pl.ds(j * H4, H4), :].astype(jnp.float32)
            out_ref[bf, pl.ds(koff, H4), :] = acc
            cb_ref[bf, pl.ds(koff, H4), :] = acc.astype(jnp.bfloat16)

        def s3_start(bf, j):
            ka, p1, p2 = roles[bf]
            off = ka * H2 + j * H4
            return start(bf, 3 + j, p1, off, H4,
                         cb_ref.at[bf, pl.ds(off, H4)])

        def s3_fin(bf, j, rdma):
            ka, _, _ = roles[bf]
            rdma.wait()
            off = (1 - ka) * H2 + j * H4
            out_ref[bf, pl.ds(off, H4), :] = cb_ref[
                bf, pl.ds(off, H4), :].astype(jnp.float32)

        ka_a, _, _ = roles[0]
        ka_b, _, _ = roles[1]
        compute_half(0, (1 - ka_a) * H2, True)
        a1 = s1_start(0)
        compute_half(1, (1 - ka_b) * H2, True)
        b1 = s1_start(1)
        compute_half(0, ka_a * H2, False)
        s1_fin(0, a1)
        a2_0 = s2_start(0, 0)
        a2_1 = s2_start(0, 1)
        compute_half(1, ka_b * H2, False)
        s1_fin(1, b1)
        b2_0 = s2_start(1, 0)
        b2_1 = s2_start(1, 1)
        s2_fin(0, 0, a2_0)
        a3_0 = s3_start(0, 0)
        s2_fin(0, 1, a2_1)
        a3_1 = s3_start(0, 1)
        s2_fin(1, 0, b2_0)
        b3_0 = s3_start(1, 0)
        s2_fin(1, 1, b2_1)
        b3_1 = s3_start(1, 1)
        s3_fin(0, 0, a3_0)
        s3_fin(0, 1, a3_1)
        s3_fin(1, 0, b3_0)
        s3_fin(1, 1, b3_1)

    return pl.pallas_call(
        body,
        out_shape=jax.ShapeDtypeStruct((B, Sq, D), jnp.float32),
        in_specs=[pl.BlockSpec(memory_space=pltpu.VMEM)] * 5,
        out_specs=pl.BlockSpec(memory_space=pltpu.VMEM),
        scratch_shapes=[
            pltpu.VMEM((H2, Hl * Dh), jnp.float32),
            pltpu.VMEM((Sq, Skv), jnp.float32),
            pltpu.VMEM((B, Sq, D), jnp.bfloat16),
            pltpu.VMEM((B, H2, D), jnp.bfloat16),
            pltpu.VMEM((B, H2, D), jnp.bfloat16),
            pltpu.SemaphoreType.DMA((B, 5)),
            pltpu.SemaphoreType.DMA((B, 5)),
        ],
        compiler_params=pltpu.CompilerParams(collective_id=0),
    )(x, Wq, K_loc, V_loc, Wo)


# device time: 33041 ns/iter; 1.0913x vs baseline; 1.0140x over previous
import jax
import jax.numpy as jnp
from jax import lax
from jax.experimental import pallas as pl
from jax.experimental.pallas import tpu as pltpu

N_DEV = 4


def kernel(x, Wq, K_ext, V_ext, Wo):
    B, Sq, D = x.shape
    Skv = K_ext.shape[1]
    Hq = K_ext.shape[2]
    Dh = K_ext.shape[3]
    Hl = Hq // N_DEV
    assert B == 2
    assert Wq.shape == (D, Hl * Dh)
    assert Wo.shape == (Hl * Dh, D)
    H2 = Sq // 2

    my = lax.axis_index("i")
    K_loc = lax.dynamic_slice_in_dim(
        K_ext.reshape(B, Skv, Hq * Dh), my * Hl * Dh, Hl * Dh, axis=2)
    V_loc = lax.dynamic_slice_in_dim(
        V_ext.reshape(B, Skv, Hq * Dh), my * Hl * Dh, Hl * Dh, axis=2)

    def body(x_ref, wq_ref, k_ref, v_ref, wo_ref, out_ref,
             ctx_ref, bias_ref, cb_ref, r1_ref, r2_ref,
             send_sems, recv_sems):
        pos = lax.axis_index("i")
        left = lax.rem(pos + N_DEV - 1, N_DEV)
        right = lax.rem(pos + 1, N_DEV)

        barrier_sem = pltpu.get_barrier_semaphore()
        for nbr in (left, right):
            pl.semaphore_signal(
                barrier_sem, inc=1,
                device_id=(nbr,), device_id_type=pl.DeviceIdType.MESH,
            )
        ri = lax.broadcasted_iota(jnp.int32, (Sq, Skv), 0)
        ci = lax.broadcasted_iota(jnp.int32, (Sq, Skv), 1)
        bias_ref[...] = jnp.where(
            ((ri // 64) % 4) == ((ci // 64) % 4), 0.0, -1e9
        ).astype(jnp.float32)
        pl.semaphore_wait(barrier_sem, 2)

        hi = pos // 2
        gray = (pos + hi) % 2
        x1 = jnp.bitwise_xor(pos, 1)
        x3 = jnp.bitwise_xor(pos, 3)
        roles = {0: (gray, x1, x3), 1: (hi, x3, x1)}

        def compute_half(b, roff, write_cb, nrows=H2):
            qh = jnp.dot(x_ref[b, pl.ds(roff, nrows), :], wq_ref[...],
                         preferred_element_type=jnp.float32)
            for h in range(Hl):
                q = qh[:, h * Dh:(h + 1) * Dh]
                k = k_ref[b][:, h * Dh:(h + 1) * Dh]
                s = lax.dot_general(
                    q, k, (((1,), (1,)), ((), ())),
                    preferred_element_type=jnp.float32)
                w = jnp.exp(s * 0.125 + bias_ref[pl.ds(roff, nrows), :])
                rs = 1.0 / jnp.sum(w, axis=1, keepdims=True)
                ctx_ref[0:nrows, h * Dh:(h + 1) * Dh] = rs * jnp.dot(
                    w, v_ref[b][:, h * Dh:(h + 1) * Dh],
                    preferred_element_type=jnp.float32)
            res = jnp.dot(ctx_ref[0:nrows, :], wo_ref[...],
                          preferred_element_type=jnp.float32)
            out_ref[b, pl.ds(roff, nrows), :] = res
            if write_cb:
                cb_ref[b, pl.ds(roff, nrows), :] = res.astype(jnp.bfloat16)

        H4 = H2 // 2

        def start(bf, st, partner, off, size, dst):
            rdma = pltpu.make_async_remote_copy(
                src_ref=cb_ref.at[bf, pl.ds(off, size)],
                dst_ref=dst,
                send_sem=send_sems.at[bf, st],
                recv_sem=recv_sems.at[bf, st],
                device_id=(partner,),
                device_id_type=pl.DeviceIdType.MESH,
            )
            rdma.start()
            return rdma

        def s1_start(bf):
            ka, p1, p2 = roles[bf]
            return start(bf, 0, p1, (1 - ka) * H2, H2, r1_ref.at[bf])

        def s1_fin(bf, rdma):
            ka, _, _ = roles[bf]
            rdma.wait()
            koff = ka * H2
            acc = out_ref[bf, pl.ds(koff, H2), :] + r1_ref[bf].astype(
                jnp.float32)
            out_ref[bf, pl.ds(koff, H2), :] = acc
            cb_ref[bf, pl.ds(koff, H2), :] = acc.astype(jnp.bfloat16)

        def s1_fin_q(bf, j, rdma):
            ka, _, _ = roles[bf]
            if rdma is not None:
                rdma.wait()
            koff = ka * H2 + j * H4
            acc = out_ref[bf, pl.ds(koff, H4), :] + r1_ref[
                bf, pl.ds(j * H4, H4), :].astype(jnp.float32)
            out_ref[bf, pl.ds(koff, H4), :] = acc
            cb_ref[bf, pl.ds(koff, H4), :] = acc.astype(jnp.bfloat16)

        def s2_start(bf, j):
            ka, p1, p2 = roles[bf]
            return start(bf, 1 + j, p2, ka * H2 + j * H4, H4,
                         r2_ref.at[bf, pl.ds(j * H4, H4)])

        def s2_fin(bf, j, rdma):
            ka, _, _ = roles[bf]
            rdma.wait()
            koff = ka * H2 + j * H4
            acc = out_ref[bf, pl.ds(koff, H4), :] + r2_ref[
                bf, pl.ds(j * H4, H4), :].astype(jnp.float32)
            out_ref[bf, pl.ds(koff, H4), :] = acc
            cb_ref[bf, pl.ds(koff, H4), :] = acc.astype(jnp.bfloat16)

        def s3_start(bf, j):
            ka, p1, p2 = roles[bf]
            off = ka * H2 + j * H4
            return start(bf, 3 + j, p1, off, H4,
                         cb_ref.at[bf, pl.ds(off, H4)])

        def s3_fin(bf, j, rdma):
            ka, _, _ = roles[bf]
            rdma.wait()
            off = (1 - ka) * H2 + j * H4
            out_ref[bf, pl.ds(off, H4), :] = cb_ref[
                bf, pl.ds(off, H4), :].astype(jnp.float32)

        ka_a, _, _ = roles[0]
        ka_b, _, _ = roles[1]
        compute_half(0, (1 - ka_a) * H2, True)
        a1 = s1_start(0)
        compute_half(1, (1 - ka_b) * H2, True)
        b1 = s1_start(1)
        compute_half(0, ka_a * H2, False)
        s1_fin(0, a1)
        a2_0 = s2_start(0, 0)
        a2_1 = s2_start(0, 1)
        compute_half(1, ka_b * H2, False, H4)
        s1_fin_q(1, 0, b1)
        b2_0 = s2_start(1, 0)
        compute_half(1, ka_b * H2 + H4, False, H4)
        s1_fin_q(1, 1, None)
        b2_1 = s2_start(1, 1)
        s2_fin(0, 0, a2_0)
        a3_0 = s3_start(0, 0)
        s2_fin(0, 1, a2_1)
        a3_1 = s3_start(0, 1)
        s2_fin(1, 0, b2_0)
        b3_0 = s3_start(1, 0)
        s2_fin(1, 1, b2_1)
        b3_1 = s3_start(1, 1)
        s3_fin(0, 0, a3_0)
        s3_fin(0, 1, a3_1)
        s3_fin(1, 0, b3_0)
        s3_fin(1, 1, b3_1)

    return pl.pallas_call(
        body,
        out_shape=jax.ShapeDtypeStruct((B, Sq, D), jnp.float32),
        in_specs=[pl.BlockSpec(memory_space=pltpu.VMEM)] * 5,
        out_specs=pl.BlockSpec(memory_space=pltpu.VMEM),
        scratch_shapes=[
            pltpu.VMEM((H2, Hl * Dh), jnp.float32),
            pltpu.VMEM((Sq, Skv), jnp.float32),
            pltpu.VMEM((B, Sq, D), jnp.bfloat16),
            pltpu.VMEM((B, H2, D), jnp.bfloat16),
            pltpu.VMEM((B, H2, D), jnp.bfloat16),
            pltpu.SemaphoreType.DMA((B, 5)),
            pltpu.SemaphoreType.DMA((B, 5)),
        ],
        compiler_params=pltpu.CompilerParams(collective_id=0),
    )(x, Wq, K_loc, V_loc, Wo)
